# baseline (speedup 1.0000x reference)
import functools

import jax
import jax.numpy as jnp
from jax import lax
from jax.experimental import pallas as pl
from jax.experimental.pallas import tpu as pltpu
from jax.experimental.pallas import tpu_sc as plsc

N = 10000
N_PAD = 10240
E = 320000
D_IN = 128
D_OUT = 128

NC = 2
NS = 16
NT = NC * NS
LANES = 16

CHUNK = 64
E_PAD = 327680
EDGES_PER_TILE = E_PAD // NT
NCHUNKS = EDGES_PER_TILE // CHUNK
ROWS_PER_TILE = N_PAD // NS
WBLK = 128

MM_BLK = 1024


def _matmul_body(x_ref, w_ref, o_ref):
    o_ref[...] = jnp.dot(x_ref[...], w_ref[...], preferred_element_type=jnp.float32)


def _tc_matmul(x, W):
    return pl.pallas_call(
        _matmul_body,
        grid=(N_PAD // MM_BLK,),
        in_specs=[
            pl.BlockSpec((MM_BLK, D_IN), lambda i: (i, 0)),
            pl.BlockSpec((D_IN, D_OUT), lambda i: (0, 0)),
        ],
        out_specs=pl.BlockSpec((MM_BLK, D_OUT), lambda i: (i, 0)),
        out_shape=jax.ShapeDtypeStruct((N_PAD, D_OUT), jnp.float32),
    )(x, W)


def _add_body(a_ref, b_ref, o_ref):
    o_ref[...] = a_ref[...] + b_ref[...]


def _tc_add(a, b):
    return pl.pallas_call(
        _add_body,
        grid=(N_PAD // MM_BLK,),
        in_specs=[
            pl.BlockSpec((MM_BLK, D_OUT), lambda i: (i, 0)),
            pl.BlockSpec((MM_BLK, D_OUT), lambda i: (i, 0)),
        ],
        out_specs=pl.BlockSpec((MM_BLK, D_OUT), lambda i: (i, 0)),
        out_shape=jax.ShapeDtypeStruct((N_PAD, D_OUT), jnp.float32),
    )(a, b)


def _bcast_lane(v, i):
    idx = jnp.full((LANES,), i, dtype=jnp.int32)
    return lax.gather(
        v,
        idx[:, None],
        dimension_numbers=lax.GatherDimensionNumbers(
            offset_dims=(), collapsed_slice_dims=(0,), start_index_map=(0,)
        ),
        slice_sizes=(1,),
        mode=lax.GatherScatterMode.PROMISE_IN_BOUNDS,
    )


def _sc_spmm_body(
    ps, rows_hbm, cols_hbm, vals_hbm, out0, out1,
    rows_all, colsA, colsB, valsA, valsB, bufA, bufB, acc,
    psem, csemA, csemB, isemA, isemB, gsemA, gsemB, ssemA, ssemB, wsem,
):
    c = lax.axis_index("c")
    s = lax.axis_index("s")
    tid = c * NS + s

    pltpu.async_copy(rows_hbm.at[tid], rows_all, psem)

    zero16 = jnp.zeros((LANES,), jnp.float32)

    def zrow(i, carry):
        for j in range(D_OUT // LANES):
            bufA[i, pl.ds(j * LANES, LANES)] = zero16
        return carry

    lax.fori_loop(0, CHUNK, zrow, 0)
    row0 = s * ROWS_PER_TILE
    for b in range(ROWS_PER_TILE // CHUNK):
        pltpu.async_copy(bufA, acc.at[pl.ds(row0 + b * CHUNK, CHUNK)], wsem)
    for b in range(ROWS_PER_TILE // CHUNK):
        pltpu.make_async_copy(bufA, acc.at[pl.ds(row0, CHUNK)], wsem).wait()
    plsc.subcore_barrier()

    def cols_load(i, cols, csem):
        pltpu.async_copy(cols_hbm.at[tid, i], cols, csem)

    def cols_wait(cols, csem):
        pltpu.make_async_copy(cols_hbm.at[0, 0], cols, csem).wait()

    def vals_load(i, vals, isem):
        pltpu.async_copy(vals_hbm.at[tid, i], vals, isem)

    def vals_wait(vals, isem):
        pltpu.make_async_copy(vals_hbm.at[0, 0], vals, isem).wait()

    def gather(cols, buf, gsem):
        pltpu.async_copy(ps.at[cols], buf, gsem)

    def gather_wait(buf, gsem):
        pltpu.make_async_copy(ps.at[pl.ds(0, CHUNK)], buf, gsem).wait()

    def scatter(i, buf, ssem):
        pltpu.async_copy(buf, acc.at[rows_all.at[i]], ssem, add=True)

    def scatter_wait(buf, ssem):
        pltpu.make_async_copy(buf, acc.at[pl.ds(0, CHUNK)], ssem).wait()

    def scale(buf, vals):
        for g in range(CHUNK // LANES):
            vv = vals[pl.ds(g * LANES, LANES)]
            for i in range(LANES):
                e = g * LANES + i
                vb = _bcast_lane(vv, i)
                for j in range(D_OUT // LANES):
                    sl = pl.ds(j * LANES, LANES)
                    buf[e, sl] = buf[e, sl] * vb

    clamp = NCHUNKS - 1

    cols_load(0, colsA, csemA)
    cols_load(1, colsB, csemB)
    vals_load(0, valsA, isemA)
    vals_load(1, valsB, isemB)
    pltpu.make_async_copy(rows_hbm.at[0], rows_all, psem).wait()
    cols_wait(colsA, csemA)
    gather(colsA, bufA, gsemA)
    cols_wait(colsB, csemB)
    gather(colsB, bufB, gsemB)

    vals_wait(valsA, isemA)
    gather_wait(bufA, gsemA)
    cols_load(2, colsA, csemA)
    vals_load(2, valsA, isemA)
    scale(bufA, valsA)
    scatter(0, bufA, ssemA)

    def half(i, colsX, valsX, bufX, csemX, isemX, gsemX, ssemX,
             colsY, bufY, csemY, gsemY, ssemY):
        scatter_wait(bufY, ssemY)
        cols_wait(colsY, csemY)
        gather(colsY, bufY, gsemY)
        vals_wait(valsX, isemX)
        gather_wait(bufX, gsemX)
        cols_load(jnp.minimum(i + 2, clamp), colsX, csemX)
        vals_load(jnp.minimum(i + 2, clamp), valsX, isemX)
        scale(bufX, valsX)
        scatter(i, bufX, ssemX)

    def body(k, carry):
        i0 = 2 * k + 1
        half(i0, colsB, valsB, bufB, csemB, isemB, gsemB, ssemB,
             colsA, bufA, csemA, gsemA, ssemA)
        half(i0 + 1, colsA, valsA, bufA, csemA, isemA, gsemA, ssemA,
             colsB, bufB, csemB, gsemB, ssemB)
        return carry

    lax.fori_loop(0, NCHUNKS // 2 - 1, body, 0)

    scatter_wait(bufA, ssemA)
    vals_wait(valsB, isemB)
    gather_wait(bufB, gsemB)
    scale(bufB, valsB)
    scatter(NCHUNKS - 1, bufB, ssemB)
    scatter_wait(bufB, ssemB)
    cols_wait(colsA, csemA)
    vals_wait(valsA, isemA)
    plsc.subcore_barrier()

    @pl.when(c == 0)
    def _():
        for b in range(ROWS_PER_TILE // WBLK):
            r = row0 + b * WBLK
            pltpu.async_copy(acc.at[pl.ds(r, WBLK)], out0.at[pl.ds(r, WBLK)], wsem)
        for b in range(ROWS_PER_TILE // WBLK):
            pltpu.make_async_copy(acc.at[pl.ds(row0, WBLK)], out0.at[pl.ds(row0, WBLK)], wsem).wait()

    @pl.when(c == 1)
    def _():
        for b in range(ROWS_PER_TILE // WBLK):
            r = row0 + b * WBLK
            pltpu.async_copy(acc.at[pl.ds(r, WBLK)], out1.at[pl.ds(r, WBLK)], wsem)
        for b in range(ROWS_PER_TILE // WBLK):
            pltpu.make_async_copy(acc.at[pl.ds(row0, WBLK)], out1.at[pl.ds(row0, WBLK)], wsem).wait()


_sc_spmm = functools.partial(
    pl.kernel,
    mesh=plsc.VectorSubcoreMesh(core_axis_name="c", subcore_axis_name="s"),
    out_type=[
        jax.ShapeDtypeStruct((N_PAD, D_OUT), jnp.float32),
        jax.ShapeDtypeStruct((N_PAD, D_OUT), jnp.float32),
    ],
    scratch_types=[
        pltpu.VMEM((NCHUNKS, CHUNK), jnp.int32),
        pltpu.VMEM((CHUNK,), jnp.int32),
        pltpu.VMEM((CHUNK,), jnp.int32),
        pltpu.VMEM((CHUNK,), jnp.float32),
        pltpu.VMEM((CHUNK,), jnp.float32),
        pltpu.VMEM((CHUNK, D_OUT), jnp.float32),
        pltpu.VMEM((CHUNK, D_OUT), jnp.float32),
        pltpu.VMEM_SHARED((N_PAD, D_OUT), jnp.float32),
        pltpu.SemaphoreType.DMA,
        pltpu.SemaphoreType.DMA,
        pltpu.SemaphoreType.DMA,
        pltpu.SemaphoreType.DMA,
        pltpu.SemaphoreType.DMA,
        pltpu.SemaphoreType.DMA,
        pltpu.SemaphoreType.DMA,
        pltpu.SemaphoreType.DMA,
        pltpu.SemaphoreType.DMA,
        pltpu.SemaphoreType.DMA,
    ],
)(_sc_spmm_body)


def kernel(x, adj_indices, adj_values, W):
    x_pad = jnp.pad(x, ((0, N_PAD - N), (0, 0)))
    ps = _tc_matmul(x_pad, W)
    rows = jnp.pad(adj_indices[0], (0, E_PAD - E)).reshape(NT, NCHUNKS, CHUNK)
    cols = jnp.pad(adj_indices[1], (0, E_PAD - E)).reshape(NT, NCHUNKS, CHUNK)
    vals = jnp.pad(adj_values, (0, E_PAD - E)).reshape(NT, NCHUNKS, CHUNK)
    p0, p1 = _sc_spmm(ps, rows, cols, vals)
    return _tc_add(p0, p1)[:N]
